# baseline (speedup 1.0000x reference)
import functools

import jax
import jax.numpy as jnp
from jax import lax
from jax.experimental import pallas as pl
from jax.experimental.pallas import tpu as pltpu
from jax.experimental.pallas import tpu_sc as plsc

NC = 2
NS = 16
GCH = 128
PKB = 14


def _make_agg(n_pad, d, e, rel, win):
    eps = e // NS
    dep = 3 if d >= 128 else 4
    cap = eps + dep * GCH
    nrow = (cap + GCH - 1) // GCH
    half = n_pad // NC
    nwin = half // win
    rpt = win // NS
    ch = eps // 10 if eps % 160 == 0 else eps
    zr = 32
    mesh = plsc.VectorSubcoreMesh(
        core_axis_name="c", subcore_axis_name="s", num_cores=NC, num_subcores=NS
    )

    @functools.partial(
        pl.kernel,
        out_type=jax.ShapeDtypeStruct((n_pad, d), jnp.float32),
        mesh=mesh,
        scratch_types=[
            pltpu.VMEM((ch,), jnp.int32),
            pltpu.VMEM((ch,), jnp.int32),
            pltpu.VMEM((ch,), jnp.int32),
            pltpu.VMEM((nrow, GCH), jnp.int32),
            pltpu.VMEM((nrow, GCH), jnp.int32),
            [pltpu.VMEM((GCH, d), jnp.float32) for _ in range(dep)],
            pltpu.VMEM((zr, d), jnp.float32),
            pltpu.VMEM_SHARED((win + 16, d), jnp.float32),
            [pltpu.SemaphoreType.DMA for _ in range(2 * dep)],
        ],
        compiler_params=pltpu.CompilerParams(
            needs_layout_passes=False,
            use_tc_tiling_on_sc=(d % 128 == 0)),
    )
    def k(h_hbm, src_hbm, dst_hbm, typ_hbm, out_hbm,
          src_raw, dst_raw, typ_raw, src2d, dst2d,
          rows, zbuf, agg, sems):
        c = lax.axis_index("c")
        s = lax.axis_index("s")
        base = s * eps
        sg, ss = sems[:dep], sems[dep:]

        zvec = jnp.zeros((16,), jnp.float32)

        def zrow(i, carry):
            for kk in range(d // 16):
                zbuf[i, pl.ds(kk * 16, 16)] = zvec
            return carry

        lax.fori_loop(0, zr, zrow, 0)

        iota16 = lax.broadcasted_iota(jnp.int32, (16,), 0)

        for w in range(nwin):
            lo = c * half + w * win

            with jax.named_scope("zero_win"):
                for z in range(rpt // zr):
                    pltpu.sync_copy(zbuf,
                                    agg.at[pl.ds(s * rpt + z * zr, zr)])

            def compact(i, cnt):
                t = typ_raw[pl.ds(i * 16, 16)]
                sv = src_raw[pl.ds(i * 16, 16)] - lo
                dv = dst_raw[pl.ds(i * 16, 16)]
                m = (t == rel) & (sv >= 0) & (sv < win)
                inc = plsc.cumsum(m.astype(jnp.int32))
                pos = cnt + inc - 1
                ph = lax.shift_right_logical(pos, 7)
                plo = pos & (GCH - 1)
                plsc.store_scatter(src2d, [ph, plo], sv, mask=m)
                plsc.store_scatter(dst2d, [ph, plo], dv, mask=m)
                return cnt + inc[15]

            cnt = jnp.int32(0)
            with jax.named_scope("compact"):
                for j in range(eps // ch):
                    pltpu.sync_copy(src_hbm.at[pl.ds(base + j * ch, ch)],
                                    src_raw)
                    pltpu.sync_copy(dst_hbm.at[pl.ds(base + j * ch, ch)],
                                    dst_raw)
                    pltpu.sync_copy(typ_hbm.at[pl.ds(base + j * ch, ch)],
                                    typ_raw)
                    cnt = lax.fori_loop(0, ch // 16, compact, cnt)

            dumsrc = jnp.full((16,), win, jnp.int32)
            zidx = jnp.zeros((16,), jnp.int32)
            for i in range(dep * GCH // 16):
                pos = cnt + i * 16 + iota16
                ph = lax.shift_right_logical(pos, 7)
                plo = pos & (GCH - 1)
                plsc.store_scatter(src2d, [ph, plo], dumsrc)
                plsc.store_scatter(dst2d, [ph, plo], zidx)

            with jax.named_scope("barrier1"):
                plsc.subcore_barrier()

            nblk = (cnt + dep * GCH - 1) // (dep * GCH)

            def blk(kk, carry):
                row = kk * dep
                gs = []
                for j in range(dep):
                    gs.append(pltpu.async_copy(
                        h_hbm.at[dst2d.at[row + j]], rows[j], sg[j]))
                sc = []
                for j in range(dep):
                    gs[j].wait()
                    sc.append(pltpu.async_copy(
                        rows[j], agg.at[src2d.at[row + j]], ss[j], add=True))
                for j in range(dep):
                    sc[j].wait()
                return carry

            with jax.named_scope("blocks"):
                lax.fori_loop(0, nblk, blk, 0)

            with jax.named_scope("barrier2"):
                plsc.subcore_barrier()
            with jax.named_scope("flush"):
                pltpu.sync_copy(agg.at[pl.ds(s * rpt, rpt)],
                                out_hbm.at[pl.ds(lo + s * rpt, rpt)])

    return k


def _dense(agg, h, wl_t, wc_t, bias8):
    n_pad, d = h.shape
    hid = wl_t.shape[1]
    bm = 1024

    def body(a_ref, h_ref, wl_ref, wc_ref, b_ref, o_ref):
        acc = jnp.dot(a_ref[...], wl_ref[...], preferred_element_type=jnp.float32)
        acc = acc + jnp.dot(h_ref[...], wc_ref[...],
                            preferred_element_type=jnp.float32)
        o_ref[...] = jnp.maximum(acc + b_ref[0:1, :], 0.0)

    return pl.pallas_call(
        body,
        grid=(n_pad // bm,),
        in_specs=[
            pl.BlockSpec((bm, d), lambda i: (i, 0)),
            pl.BlockSpec((bm, d), lambda i: (i, 0)),
            pl.BlockSpec((d, hid), lambda i: (0, 0)),
            pl.BlockSpec((d, hid), lambda i: (0, 0)),
            pl.BlockSpec((8, hid), lambda i: (0, 0)),
        ],
        out_specs=pl.BlockSpec((bm, hid), lambda i: (i, 0)),
        out_shape=jax.ShapeDtypeStruct((n_pad, hid), jnp.float32),
    )(agg, h, wl_t, wc_t, bias8)


def kernel(x, edge_index, edge_type, params):
    n, d0 = x.shape
    e = edge_index.shape[1]
    n_pad = ((n // 2048) + 1) * 2048
    half = n_pad // NC

    src = edge_index[0]
    dst = edge_index[1]
    h = jnp.zeros((n_pad, d0), x.dtype).at[:n].set(x)

    for rel, p in enumerate(params):
        d = h.shape[1]
        if d >= 128:
            win = 2560 if half % 2560 == 0 else 1024
        else:
            win = half
        agg = _make_agg(n_pad, d, e, rel, win)(h, src, dst, edge_type)
        wl_t = p["w_l_W"].T
        wc_t = (p["w_0_W"] + p["w_1_W"]).T
        bias = p["w_l_b"] + p["w_0_b"] + p["w_1_b"]
        bias8 = jnp.broadcast_to(bias[None, :], (8, bias.shape[0]))
        h = _dense(agg, h, wl_t, wc_t, bias8)

    return h[:n]


# speedup vs baseline: 1.1900x; 1.1900x over previous
import functools

import jax
import jax.numpy as jnp
from jax import lax
from jax.experimental import pallas as pl
from jax.experimental.pallas import tpu as pltpu
from jax.experimental.pallas import tpu_sc as plsc

NC = 2
NS = 16
GCH = 128
PKB = 14


def _make_agg(n_pad, d, e, rel, win):
    eps = e // NS
    bf = d % 128 == 0
    dep = 3 if d >= 128 else 4
    cap = eps + dep * GCH
    nrow = (cap + GCH - 1) // GCH
    half = n_pad // NC
    nwin = half // win
    rpt = win // NS
    ch = eps // 10 if eps % 160 == 0 else eps
    zr = 32
    mesh = plsc.VectorSubcoreMesh(
        core_axis_name="c", subcore_axis_name="s", num_cores=NC, num_subcores=NS
    )

    @functools.partial(
        pl.kernel,
        out_type=jax.ShapeDtypeStruct((n_pad, d), jnp.float32),
        mesh=mesh,
        scratch_types=[
            pltpu.VMEM((ch,), jnp.int32),
            pltpu.VMEM((ch,), jnp.int32),
            pltpu.VMEM((ch,), jnp.int32),
            pltpu.VMEM((nrow, GCH), jnp.int32),
            pltpu.VMEM((nrow, GCH), jnp.int32),
            [pltpu.VMEM((GCH, d), jnp.bfloat16 if bf else jnp.float32)
             for _ in range(dep)],
            [pltpu.VMEM((GCH, d), jnp.float32) for _ in range(2 if bf else 0)],
            pltpu.VMEM((zr, d), jnp.float32),
            pltpu.VMEM_SHARED((win + 16, d), jnp.float32),
            [pltpu.SemaphoreType.DMA for _ in range(2 * dep)],
        ],
        compiler_params=pltpu.CompilerParams(
            needs_layout_passes=False,
            use_tc_tiling_on_sc=False),
    )
    def k(h_hbm, src_hbm, dst_hbm, typ_hbm, out_hbm,
          src_raw, dst_raw, typ_raw, src2d, dst2d,
          rows, rowsf, zbuf, agg, sems):
        c = lax.axis_index("c")
        s = lax.axis_index("s")
        base = s * eps
        sg, ss = sems[:dep], sems[dep:]

        zvec = jnp.zeros((16,), jnp.float32)

        def zrow(i, carry):
            for kk in range(d // 16):
                zbuf[i, pl.ds(kk * 16, 16)] = zvec
            return carry

        lax.fori_loop(0, zr, zrow, 0)

        iota16 = lax.broadcasted_iota(jnp.int32, (16,), 0)

        for w in range(nwin):
            lo = c * half + w * win

            with jax.named_scope("zero_win"):
                for z in range(rpt // zr):
                    pltpu.sync_copy(zbuf,
                                    agg.at[pl.ds(s * rpt + z * zr, zr)])

            def compact(i, cnt):
                t = typ_raw[pl.ds(i * 16, 16)]
                sv = src_raw[pl.ds(i * 16, 16)] - lo
                dv = dst_raw[pl.ds(i * 16, 16)]
                m = (t == rel) & (sv >= 0) & (sv < win)
                inc = plsc.cumsum(m.astype(jnp.int32))
                pos = cnt + inc - 1
                ph = lax.shift_right_logical(pos, 7)
                plo = pos & (GCH - 1)
                plsc.store_scatter(src2d, [ph, plo], sv, mask=m)
                plsc.store_scatter(dst2d, [ph, plo], dv, mask=m)
                return cnt + inc[15]

            cnt = jnp.int32(0)
            with jax.named_scope("compact"):
                for j in range(eps // ch):
                    pltpu.sync_copy(src_hbm.at[pl.ds(base + j * ch, ch)],
                                    src_raw)
                    pltpu.sync_copy(dst_hbm.at[pl.ds(base + j * ch, ch)],
                                    dst_raw)
                    pltpu.sync_copy(typ_hbm.at[pl.ds(base + j * ch, ch)],
                                    typ_raw)
                    cnt = lax.fori_loop(0, ch // 16, compact, cnt)

            dumsrc = jnp.full((16,), win, jnp.int32)
            zidx = jnp.zeros((16,), jnp.int32)
            for i in range(dep * GCH // 16):
                pos = cnt + i * 16 + iota16
                ph = lax.shift_right_logical(pos, 7)
                plo = pos & (GCH - 1)
                plsc.store_scatter(src2d, [ph, plo], dumsrc)
                plsc.store_scatter(dst2d, [ph, plo], zidx)

            with jax.named_scope("barrier1"):
                plsc.subcore_barrier()

            nblk = (cnt + dep * GCH - 1) // (dep * GCH)

            def conv(src_ref, dst_ref):
                def crow(i, carry):
                    for kk in range(d // 32):
                        v = plsc.bitcast(src_ref[i, pl.ds(kk * 32, 32)],
                                         jnp.int32)
                        lo = plsc.bitcast(lax.shift_left(v, 16), jnp.float32)
                        hi = plsc.bitcast(v & jnp.int32(-65536), jnp.float32)
                        dst_ref[i, pl.ds(kk * 32, 16)] = lo
                        dst_ref[i, pl.ds(kk * 32 + 16, 16)] = hi
                    return carry

                lax.fori_loop(0, GCH, crow, 0)

            def blk(kk, carry):
                row = kk * dep
                gs = []
                for j in range(dep):
                    gs.append(pltpu.async_copy(
                        h_hbm.at[dst2d.at[row + j]], rows[j], sg[j]))
                sc = {}
                for j in range(dep):
                    gs[j].wait()
                    if bf:
                        if j >= 2:
                            sc[j - 2].wait()
                        conv(rows[j], rowsf[j % 2])
                        sc[j] = pltpu.async_copy(
                            rowsf[j % 2], agg.at[src2d.at[row + j]],
                            ss[j % 2], add=True)
                    else:
                        sc[j] = pltpu.async_copy(
                            rows[j], agg.at[src2d.at[row + j]], ss[j],
                            add=True)
                for j in range(max(0, dep - 2) if bf else 0, dep):
                    sc[j].wait()
                return carry

            with jax.named_scope("blocks"):
                lax.fori_loop(0, nblk, blk, 0)

            with jax.named_scope("barrier2"):
                plsc.subcore_barrier()
            with jax.named_scope("flush"):
                pltpu.sync_copy(agg.at[pl.ds(s * rpt, rpt)],
                                out_hbm.at[pl.ds(lo + s * rpt, rpt)])

    return k


def _dense(agg, h, wl_t, wc_t, bias8):
    n_pad, d = h.shape
    hid = wl_t.shape[1]
    bm = 1024

    def body(a_ref, h_ref, wl_ref, wc_ref, b_ref, o_ref):
        acc = jnp.dot(a_ref[...], wl_ref[...], preferred_element_type=jnp.float32)
        acc = acc + jnp.dot(h_ref[...], wc_ref[...],
                            preferred_element_type=jnp.float32)
        o_ref[...] = jnp.maximum(acc + b_ref[0:1, :], 0.0)

    return pl.pallas_call(
        body,
        grid=(n_pad // bm,),
        in_specs=[
            pl.BlockSpec((bm, d), lambda i: (i, 0)),
            pl.BlockSpec((bm, d), lambda i: (i, 0)),
            pl.BlockSpec((d, hid), lambda i: (0, 0)),
            pl.BlockSpec((d, hid), lambda i: (0, 0)),
            pl.BlockSpec((8, hid), lambda i: (0, 0)),
        ],
        out_specs=pl.BlockSpec((bm, hid), lambda i: (i, 0)),
        out_shape=jax.ShapeDtypeStruct((n_pad, hid), jnp.float32),
    )(agg, h, wl_t, wc_t, bias8)


def kernel(x, edge_index, edge_type, params):
    n, d0 = x.shape
    e = edge_index.shape[1]
    n_pad = ((n // 2048) + 1) * 2048
    half = n_pad // NC

    src = edge_index[0]
    dst = edge_index[1]
    h = jnp.zeros((n_pad, d0), x.dtype).at[:n].set(x)

    for rel, p in enumerate(params):
        d = h.shape[1]
        if d >= 128:
            win = 2560 if half % 2560 == 0 else 1024
        else:
            win = half
        bf = d % 128 == 0
        h_in = h.astype(jnp.bfloat16) if bf else h
        agg = _make_agg(n_pad, d, e, rel, win)(h_in, src, dst, edge_type)
        wl_t = p["w_l_W"].T
        if bf:
            perm = []
            for g in range(d // 32):
                perm += [g * 32 + 2 * kk for kk in range(16)]
                perm += [g * 32 + 2 * kk + 1 for kk in range(16)]
            wl_t = wl_t[jnp.array(perm, jnp.int32), :]
        wc_t = (p["w_0_W"] + p["w_1_W"]).T
        bias = p["w_l_b"] + p["w_0_b"] + p["w_1_b"]
        bias8 = jnp.broadcast_to(bias[None, :], (8, bias.shape[0]))
        h = _dense(agg, h, wl_t, wc_t, bias8)

    return h[:n]


# speedup vs baseline: 1.3449x; 1.1302x over previous
import functools

import jax
import jax.numpy as jnp
from jax import lax
from jax.experimental import pallas as pl
from jax.experimental.pallas import tpu as pltpu
from jax.experimental.pallas import tpu_sc as plsc

NC = 2
NS = 16
GCH = 128
PKB = 14


def _make_agg(n_pad, d, e, rel, win):
    eps = e // NS
    bf = d % 32 == 0
    dep = 3 if d >= 128 else 4
    cap = eps + dep * GCH
    nrow = (cap + GCH - 1) // GCH
    half = n_pad // NC
    nwin = half // win
    rpt = win // NS
    ch = eps // 10 if eps % 160 == 0 else eps
    zr = 32
    mesh = plsc.VectorSubcoreMesh(
        core_axis_name="c", subcore_axis_name="s", num_cores=NC, num_subcores=NS
    )

    @functools.partial(
        pl.kernel,
        out_type=jax.ShapeDtypeStruct((n_pad, d), jnp.float32),
        mesh=mesh,
        scratch_types=[
            pltpu.VMEM((ch,), jnp.int32),
            pltpu.VMEM((ch,), jnp.int32),
            pltpu.VMEM((ch,), jnp.int32),
            pltpu.VMEM((nrow, GCH), jnp.int32),
            pltpu.VMEM((nrow, GCH), jnp.int32),
            [pltpu.VMEM((GCH, d), jnp.bfloat16 if bf else jnp.float32)
             for _ in range(dep)],
            [pltpu.VMEM((GCH, d), jnp.float32) for _ in range(2 if bf else 0)],
            pltpu.VMEM((zr, d), jnp.float32),
            pltpu.VMEM_SHARED((win + 16, d), jnp.float32),
            [pltpu.SemaphoreType.DMA for _ in range(2 * dep)],
        ],
        compiler_params=pltpu.CompilerParams(
            needs_layout_passes=False,
            use_tc_tiling_on_sc=False),
    )
    def k(h_hbm, src_hbm, dst_hbm, typ_hbm, out_hbm,
          src_raw, dst_raw, typ_raw, src2d, dst2d,
          rows, rowsf, zbuf, agg, sems):
        c = lax.axis_index("c")
        s = lax.axis_index("s")
        base = s * eps
        sg, ss = sems[:dep], sems[dep:]

        zvec = jnp.zeros((16,), jnp.float32)

        def zrow(i, carry):
            for kk in range(d // 16):
                zbuf[i, pl.ds(kk * 16, 16)] = zvec
            return carry

        lax.fori_loop(0, zr, zrow, 0)

        iota16 = lax.broadcasted_iota(jnp.int32, (16,), 0)

        for w in range(nwin):
            lo = c * half + w * win

            with jax.named_scope("zero_win"):
                for z in range(rpt // zr):
                    pltpu.sync_copy(zbuf,
                                    agg.at[pl.ds(s * rpt + z * zr, zr)])

            def compact(i, cnt):
                t = typ_raw[pl.ds(i * 16, 16)]
                sv = src_raw[pl.ds(i * 16, 16)] - lo
                dv = dst_raw[pl.ds(i * 16, 16)]
                m = (t == rel) & (sv >= 0) & (sv < win)
                inc = plsc.cumsum(m.astype(jnp.int32))
                pos = cnt + inc - 1
                ph = lax.shift_right_logical(pos, 7)
                plo = pos & (GCH - 1)
                plsc.store_scatter(src2d, [ph, plo], sv, mask=m)
                plsc.store_scatter(dst2d, [ph, plo], dv, mask=m)
                return cnt + inc[15]

            cnt = jnp.int32(0)
            with jax.named_scope("compact"):
                for j in range(eps // ch):
                    pltpu.sync_copy(src_hbm.at[pl.ds(base + j * ch, ch)],
                                    src_raw)
                    pltpu.sync_copy(dst_hbm.at[pl.ds(base + j * ch, ch)],
                                    dst_raw)
                    pltpu.sync_copy(typ_hbm.at[pl.ds(base + j * ch, ch)],
                                    typ_raw)
                    cnt = lax.fori_loop(0, ch // 16, compact, cnt)

            dumsrc = jnp.full((16,), win, jnp.int32)
            zidx = jnp.zeros((16,), jnp.int32)
            for i in range(dep * GCH // 16):
                pos = cnt + i * 16 + iota16
                ph = lax.shift_right_logical(pos, 7)
                plo = pos & (GCH - 1)
                plsc.store_scatter(src2d, [ph, plo], dumsrc)
                plsc.store_scatter(dst2d, [ph, plo], zidx)

            with jax.named_scope("barrier1"):
                plsc.subcore_barrier()

            nblk = (cnt + dep * GCH - 1) // (dep * GCH)

            def conv(src_ref, dst_ref):
                def crow(i, carry):
                    for kk in range(d // 32):
                        v = plsc.bitcast(src_ref[i, pl.ds(kk * 32, 32)],
                                         jnp.int32)
                        lo = plsc.bitcast(lax.shift_left(v, 16), jnp.float32)
                        hi = plsc.bitcast(v & jnp.int32(-65536), jnp.float32)
                        dst_ref[i, pl.ds(kk * 32, 16)] = lo
                        dst_ref[i, pl.ds(kk * 32 + 16, 16)] = hi
                    return carry

                lax.fori_loop(0, GCH, crow, 0)

            def blk(kk, carry):
                row = kk * dep
                gs = []
                for j in range(dep):
                    gs.append(pltpu.async_copy(
                        h_hbm.at[dst2d.at[row + j]], rows[j], sg[j]))
                sc = {}
                for j in range(dep):
                    gs[j].wait()
                    if bf:
                        if j >= 2:
                            sc[j - 2].wait()
                        conv(rows[j], rowsf[j % 2])
                        sc[j] = pltpu.async_copy(
                            rowsf[j % 2], agg.at[src2d.at[row + j]],
                            ss[j % 2], add=True)
                    else:
                        sc[j] = pltpu.async_copy(
                            rows[j], agg.at[src2d.at[row + j]], ss[j],
                            add=True)
                for j in range(max(0, dep - 2) if bf else 0, dep):
                    sc[j].wait()
                return carry

            with jax.named_scope("blocks"):
                lax.fori_loop(0, nblk, blk, 0)

            with jax.named_scope("barrier2"):
                plsc.subcore_barrier()
            with jax.named_scope("flush"):
                pltpu.sync_copy(agg.at[pl.ds(s * rpt, rpt)],
                                out_hbm.at[pl.ds(lo + s * rpt, rpt)])

    return k


def _dense(agg, h, wl_t, wc_t, bias8):
    n_pad, d = h.shape
    hid = wl_t.shape[1]
    bm = 1024

    def body(a_ref, h_ref, wl_ref, wc_ref, b_ref, o_ref):
        acc = jnp.dot(a_ref[...], wl_ref[...], preferred_element_type=jnp.float32)
        acc = acc + jnp.dot(h_ref[...], wc_ref[...],
                            preferred_element_type=jnp.float32)
        o_ref[...] = jnp.maximum(acc + b_ref[0:1, :], 0.0)

    return pl.pallas_call(
        body,
        grid=(n_pad // bm,),
        in_specs=[
            pl.BlockSpec((bm, d), lambda i: (i, 0)),
            pl.BlockSpec((bm, d), lambda i: (i, 0)),
            pl.BlockSpec((d, hid), lambda i: (0, 0)),
            pl.BlockSpec((d, hid), lambda i: (0, 0)),
            pl.BlockSpec((8, hid), lambda i: (0, 0)),
        ],
        out_specs=pl.BlockSpec((bm, hid), lambda i: (i, 0)),
        out_shape=jax.ShapeDtypeStruct((n_pad, hid), jnp.float32),
    )(agg, h, wl_t, wc_t, bias8)


def kernel(x, edge_index, edge_type, params):
    n, d0 = x.shape
    e = edge_index.shape[1]
    n_pad = ((n // 2048) + 1) * 2048
    half = n_pad // NC

    src = edge_index[0]
    dst = edge_index[1]
    h = jnp.zeros((n_pad, d0), x.dtype).at[:n].set(x)

    for rel, p in enumerate(params):
        d = h.shape[1]
        if d >= 128:
            win = 2560 if half % 2560 == 0 else 1024
        else:
            win = half
        bf = d % 32 == 0
        h_in = h.astype(jnp.bfloat16) if bf else h
        agg = _make_agg(n_pad, d, e, rel, win)(h_in, src, dst, edge_type)
        wl_t = p["w_l_W"].T
        if bf:
            perm = []
            for g in range(d // 32):
                perm += [g * 32 + 2 * kk for kk in range(16)]
                perm += [g * 32 + 2 * kk + 1 for kk in range(16)]
            wl_t = wl_t[jnp.array(perm, jnp.int32), :]
        wc_t = (p["w_0_W"] + p["w_1_W"]).T
        bias = p["w_l_b"] + p["w_0_b"] + p["w_1_b"]
        bias8 = jnp.broadcast_to(bias[None, :], (8, bias.shape[0]))
        h = _dense(agg, h, wl_t, wc_t, bias8)

    return h[:n]
